# baseline (speedup 1.0000x reference)
import jax
import jax.numpy as jnp
from jax.experimental import pallas as pl

_H, _W, _N = 480, 640, 3
_BOXES = ((50, 30, 200, 180), (120, 150, 300, 350), (400, 200, 580, 400))
_CLASS_IDS = (1, 5, 3)
_SCORES = (0.85, 0.75, 0.7)


_RB = 96


def _mask_kernel(rois_ref, masks_ref, cls_ref, scores_ref):
    g = pl.program_id(0)
    row = jax.lax.broadcasted_iota(jnp.int32, (_RB, _W), 0) + g * _RB
    col = jax.lax.broadcasted_iota(jnp.int32, (_RB, _W), 1)
    for i, (y1, x1, y2, x2) in enumerate(_BOXES):
        m = (row >= y1) & (row < y2) & (col >= x1) & (col < x2)
        masks_ref[i] = m.astype(jnp.uint8)
    @pl.when(g == 0)
    def _():
        flat = (jax.lax.broadcasted_iota(jnp.int32, (_N, 4), 0) * 4
                + jax.lax.broadcasted_iota(jnp.int32, (_N, 4), 1))
        rois = jnp.zeros((_N, 4), jnp.int32)
        for i, box in enumerate(_BOXES):
            for j, v in enumerate(box):
                rois = jnp.where(flat == i * 4 + j, jnp.int32(v), rois)
        rois_ref[...] = rois

        det = jax.lax.broadcasted_iota(jnp.int32, (_N,), 0)
        cls = jnp.zeros((_N,), jnp.int32)
        sco = jnp.zeros((_N,), jnp.float32)
        for i in range(_N):
            cls = jnp.where(det == i, jnp.int32(_CLASS_IDS[i]), cls)
            sco = jnp.where(det == i, jnp.float32(_SCORES[i]), sco)
        cls_ref[...] = cls
        scores_ref[...] = sco


def kernel(inputs, Wc, bc):
    del inputs, Wc, bc
    return pl.pallas_call(
        _mask_kernel,
        grid=(_H // _RB,),
        out_specs=(
            pl.BlockSpec((_N, 4), lambda g: (0, 0)),
            pl.BlockSpec((_N, _RB, _W), lambda g: (0, g, 0)),
            pl.BlockSpec((_N,), lambda g: (0,)),
            pl.BlockSpec((_N,), lambda g: (0,)),
        ),
        out_shape=(
            jax.ShapeDtypeStruct((_N, 4), jnp.int32),
            jax.ShapeDtypeStruct((_N, _H, _W), jnp.uint8),
            jax.ShapeDtypeStruct((_N,), jnp.int32),
            jax.ShapeDtypeStruct((_N,), jnp.float32),
        ),
    )()


# speedup vs baseline: 1.8451x; 1.8451x over previous
import jax
import jax.numpy as jnp
from jax.experimental import pallas as pl

_H, _W, _N = 480, 640, 3
_BOXES = ((50, 30, 200, 180), (120, 150, 300, 350), (400, 200, 580, 400))
_CLASS_IDS = (1, 5, 3)
_SCORES = (0.85, 0.75, 0.7)


def _mask_kernel(rois_ref, masks_ref, cls_ref, scores_ref):
    row = jax.lax.broadcasted_iota(jnp.int32, (_H, _W), 0)
    col = jax.lax.broadcasted_iota(jnp.int32, (_H, _W), 1)
    for i, (y1, x1, y2, x2) in enumerate(_BOXES):
        m = (row >= y1) & (row < y2) & (col >= x1) & (col < x2)
        masks_ref[i] = m.astype(jnp.uint8)
    flat = (jax.lax.broadcasted_iota(jnp.int32, (_N, 4), 0) * 4
            + jax.lax.broadcasted_iota(jnp.int32, (_N, 4), 1))
    rois = jnp.zeros((_N, 4), jnp.int32)
    for i, box in enumerate(_BOXES):
        for j, v in enumerate(box):
            rois = jnp.where(flat == i * 4 + j, jnp.int32(v), rois)
    rois_ref[...] = rois

    det = jax.lax.broadcasted_iota(jnp.int32, (_N,), 0)
    cls = jnp.zeros((_N,), jnp.int32)
    sco = jnp.zeros((_N,), jnp.float32)
    for i in range(_N):
        cls = jnp.where(det == i, jnp.int32(_CLASS_IDS[i]), cls)
        sco = jnp.where(det == i, jnp.float32(_SCORES[i]), sco)
    cls_ref[...] = cls
    scores_ref[...] = sco


def kernel(inputs, Wc, bc):
    del inputs, Wc, bc
    return pl.pallas_call(
        _mask_kernel,
        out_shape=(
            jax.ShapeDtypeStruct((_N, 4), jnp.int32),
            jax.ShapeDtypeStruct((_N, _H, _W), jnp.uint8),
            jax.ShapeDtypeStruct((_N,), jnp.int32),
            jax.ShapeDtypeStruct((_N,), jnp.float32),
        ),
    )()
